# baseline (speedup 1.0000x reference)
import functools

import jax
import jax.numpy as jnp
from jax import lax
from jax.experimental import pallas as pl
from jax.experimental.pallas import tpu as pltpu
from jax.experimental.pallas import tpu_sc as plsc

N = 10000
E = 320000
D = 128

NC = 2
NS = 16
NW = NC * NS
CHN = 128
NCH = 80
TCH = NW * NCH
CPT0 = 128
CPT1 = 32
QR = CHN // 4
SLAB = 64
TCHP = TCH + SLAB - CPT1
EPAD = TCHP * CHN
NROWS = 10112
RPT = NROWS // NS
DPT = 640
NPAD = NS * DPT

f32 = jnp.float32

_mesh = plsc.VectorSubcoreMesh(
    core_axis_name="c", subcore_axis_name="s", num_cores=NC, num_subcores=NS
)


def _deg_body(dst_hbm, out_hbm, didx, ones_v, zb_v, dacc, sem):
    c = lax.axis_index("c")
    s = lax.axis_index("s")
    wid = s * NC + c

    @pl.loop(0, CHN // 16)
    def _(j):
        ones_v[pl.ds(j * 16, 16)] = jnp.ones((16,), f32)

    @pl.loop(0, DPT // 16)
    def _(j):
        zb_v[pl.ds(j * 16, 16)] = jnp.zeros((16,), f32)

    pltpu.sync_copy(zb_v, dacc.at[pl.ds(s * DPT, DPT)])
    pltpu.sync_copy(dst_hbm.at[wid], didx)
    plsc.subcore_barrier()

    @pl.loop(0, NCH)
    def _(i):
        pltpu.sync_copy(ones_v, dacc.at[didx.at[i]], add=True)

    plsc.subcore_barrier()
    pltpu.sync_copy(dacc.at[pl.ds(s * DPT, DPT)],
                    out_hbm.at[c, pl.ds(s * DPT, DPT)])


@jax.jit
def _sc_degree(dst3):
    return pl.kernel(
        _deg_body,
        out_type=jax.ShapeDtypeStruct((NC, NPAD), f32),
        mesh=_mesh,
        scratch_types=[
            pltpu.VMEM((NCH, CHN), jnp.int32),
            pltpu.VMEM((CHN,), f32),
            pltpu.VMEM((DPT,), f32),
            pltpu.VMEM_SHARED((NPAD,), f32),
            pltpu.SemaphoreType.DMA,
        ],
    )(dst3)


def _segsum_body(y_hbm, src_hbm, dst_hbm, out_hbm,
                 sring, didx, rows0, rows1, acc, semf, semg0, semg1):
    c = lax.axis_index("c")
    s = lax.axis_index("s")
    rows = (rows0, rows1)
    semg = (semg0, semg1)

    base = jnp.where(c == 0, s * CPT0, NS * CPT0 + s * CPT1)
    nch = jnp.where(c == 0, CPT0, CPT1)

    @pl.loop(0, CHN)
    def _(r):
        for j in range(D // 16):
            rows0[r, pl.ds(j * 16, 16)] = jnp.zeros((16,), f32)

    pltpu.sync_copy(dst_hbm.at[pl.ds(base, SLAB)], didx)

    @pl.loop(0, RPT // CHN)
    def _(k):
        pltpu.sync_copy(rows0, acc.at[pl.ds(s * RPT + k * CHN, CHN)])

    pltpu.sync_copy(rows0.at[pl.ds(0, RPT % CHN)],
                    acc.at[pl.ds(s * RPT + (RPT // CHN) * CHN, RPT % CHN)])

    plsc.subcore_barrier()

    def fetch(i, slot):
        for q in range(4):
            pltpu.async_copy(
                src_hbm.at[pl.ds((base + i) * CHN + q * QR, QR)],
                sring.at[slot * 4 + q], semf)

    def fetch_wait(slot):
        for q in range(4):
            pltpu.make_async_copy(src_hbm.at[pl.ds(0, QR)],
                                  sring.at[slot * 4 + q], semf).wait()

    def gather(slot, k):
        for q in range(4):
            pltpu.async_copy(y_hbm.at[sring.at[slot * 4 + q]],
                             rows[k].at[pl.ds(q * QR, QR)], semg[k])

    def gather_wait(slot, k):
        for q in range(4):
            pltpu.make_async_copy(y_hbm.at[sring.at[slot * 4 + q]],
                                  rows[k].at[pl.ds(q * QR, QR)],
                                  semg[k]).wait()

    fetch(0, 0)
    fetch(1, 1)
    fetch_wait(0)
    gather(0, 0)

    @pl.loop(0, nch // 4)
    def _(g):
        for k in range(4):
            i = 4 * g + k
            nf = (k + 1) % 4
            nf2 = (k + 2) % 4

            @pl.when(i == SLAB)
            def _():
                pltpu.sync_copy(dst_hbm.at[pl.ds(base + SLAB, SLAB)], didx)

            @pl.when(i + 1 < nch)
            def _():
                fetch_wait(nf)
                gather(nf, (k + 1) % 2)

            @pl.when(i + 2 < nch)
            def _():
                fetch(i + 2, nf2)

            gather_wait(k % 4, k % 2)
            iw = jnp.where(i < SLAB, i, i - SLAB)
            pltpu.sync_copy(rows[k % 2], acc.at[didx.at[iw]], add=True)

    plsc.subcore_barrier()
    pltpu.sync_copy(acc.at[pl.ds(s * RPT, RPT)],
                    out_hbm.at[c, pl.ds(s * RPT, RPT)])


@jax.jit
def _sc_segsum(y, src2, dst2):
    return pl.kernel(
        _segsum_body,
        out_type=jax.ShapeDtypeStruct((NC, NROWS, D), f32),
        mesh=_mesh,
        scratch_types=[
            pltpu.VMEM((16, QR), jnp.int32),
            pltpu.VMEM((SLAB, CHN), jnp.int32),
            pltpu.VMEM((CHN, D), f32),
            pltpu.VMEM((CHN, D), f32),
            pltpu.VMEM_SHARED((NROWS, D), f32),
            pltpu.SemaphoreType.DMA,
            pltpu.SemaphoreType.DMA,
            pltpu.SemaphoreType.DMA,
        ],
    )(y, src2, dst2)


BR = 400
GRID = N // BR


def _s1_body(x_ref, w_ref, d0_ref, d1_ref, xw_ref, y_ref, dinv_ref):
    xw = jnp.dot(x_ref[...], w_ref[...], preferred_element_type=f32)
    dinv = lax.rsqrt(d0_ref[...] + d1_ref[...] + 1.0)
    xw_ref[...] = xw
    y_ref[...] = xw * dinv
    dinv_ref[...] = dinv


@jax.jit
def _tc_stage1(x, W1, d0, d1):
    return pl.pallas_call(
        _s1_body,
        grid=(GRID,),
        in_specs=[
            pl.BlockSpec((BR, D), lambda i: (i, 0)),
            pl.BlockSpec((D, D), lambda i: (0, 0)),
            pl.BlockSpec((BR, 1), lambda i: (i, 0)),
            pl.BlockSpec((BR, 1), lambda i: (i, 0)),
        ],
        out_specs=[
            pl.BlockSpec((BR, D), lambda i: (i, 0)),
            pl.BlockSpec((BR, D), lambda i: (i, 0)),
            pl.BlockSpec((BR, 1), lambda i: (i, 0)),
        ],
        out_shape=[
            jax.ShapeDtypeStruct((N, D), f32),
            jax.ShapeDtypeStruct((N, D), f32),
            jax.ShapeDtypeStruct((N, 1), f32),
        ],
    )(x, W1, d0, d1)


def _s2_body(s0_ref, s1_ref, xw1_ref, dinv_ref, b1_ref, w2_ref,
             xw2_ref, y2_ref):
    dinv = dinv_ref[...]
    h = (s0_ref[...] + s1_ref[...]) * dinv \
        + xw1_ref[...] * (dinv * dinv) + b1_ref[...]
    h = jnp.maximum(h, 0.0)
    xw2 = jnp.dot(h, w2_ref[...], preferred_element_type=f32)
    xw2_ref[...] = xw2
    y2_ref[...] = xw2 * dinv


@jax.jit
def _tc_stage2(s0, s1, xw1, dinv, b1, W2):
    return pl.pallas_call(
        _s2_body,
        grid=(GRID,),
        in_specs=[
            pl.BlockSpec((BR, D), lambda i: (i, 0)),
            pl.BlockSpec((BR, D), lambda i: (i, 0)),
            pl.BlockSpec((BR, D), lambda i: (i, 0)),
            pl.BlockSpec((BR, 1), lambda i: (i, 0)),
            pl.BlockSpec((1, D), lambda i: (0, 0)),
            pl.BlockSpec((D, D), lambda i: (0, 0)),
        ],
        out_specs=[
            pl.BlockSpec((BR, D), lambda i: (i, 0)),
            pl.BlockSpec((BR, D), lambda i: (i, 0)),
        ],
        out_shape=[
            jax.ShapeDtypeStruct((N, D), f32),
            jax.ShapeDtypeStruct((N, D), f32),
        ],
    )(s0, s1, xw1, dinv, b1, W2)


def _s3_body(s0_ref, s1_ref, xw2_ref, dinv_ref, b2_ref, out_ref):
    dinv = dinv_ref[...]
    out_ref[...] = (s0_ref[...] + s1_ref[...]) * dinv \
        + xw2_ref[...] * (dinv * dinv) + b2_ref[...]


@jax.jit
def _tc_stage3(s0, s1, xw2, dinv, b2):
    return pl.pallas_call(
        _s3_body,
        grid=(GRID,),
        in_specs=[
            pl.BlockSpec((BR, D), lambda i: (i, 0)),
            pl.BlockSpec((BR, D), lambda i: (i, 0)),
            pl.BlockSpec((BR, D), lambda i: (i, 0)),
            pl.BlockSpec((BR, 1), lambda i: (i, 0)),
            pl.BlockSpec((1, D), lambda i: (0, 0)),
        ],
        out_specs=pl.BlockSpec((BR, D), lambda i: (i, 0)),
        out_shape=jax.ShapeDtypeStruct((N, D), f32),
    )(s0, s1, xw2, dinv, b2)


def kernel(x, edge_index, W1, b1, W2, b2):
    src = edge_index[0].astype(jnp.int32)
    dst = edge_index[1].astype(jnp.int32)
    npad = EPAD - E
    src1d = jnp.concatenate([src, jnp.zeros((npad,), jnp.int32)])
    dst2 = jnp.concatenate(
        [dst, jnp.full((npad,), N, jnp.int32)]).reshape(TCHP, CHN)
    dst3 = dst2[:TCH].reshape(NW, NCH, CHN)

    degp = _sc_degree(dst3)
    d0 = degp[0, :N].reshape(N, 1)
    d1 = degp[1, :N].reshape(N, 1)

    xw1, y1, dinv = _tc_stage1(x, W1, d0, d1)

    s1p = _sc_segsum(y1, src1d, dst2)
    xw2, y2 = _tc_stage2(s1p[0, :N], s1p[1, :N], xw1, dinv,
                         b1.reshape(1, D), W2)

    s2p = _sc_segsum(y2, src1d, dst2)
    out = _tc_stage3(s2p[0, :N], s2p[1, :N], xw2, dinv, b2.reshape(1, D))
    return out


# speedup vs baseline: 1.0500x; 1.0500x over previous
import functools

import jax
import jax.numpy as jnp
from jax import lax
from jax.experimental import pallas as pl
from jax.experimental.pallas import tpu as pltpu
from jax.experimental.pallas import tpu_sc as plsc

N = 10000
E = 320000
D = 128

NC = 2
NS = 16
NW = NC * NS
CHN = 128
NCH = 80
TCH = NW * NCH
CPT0 = 144
CPT1 = 16
SLAB = 64
TCHP = TCH + SLAB - CPT1
EPAD = TCHP * CHN
NROWS = 10112
RPT = NROWS // NS
DPT = 640
NPAD = NS * DPT

f32 = jnp.float32

_mesh = plsc.VectorSubcoreMesh(
    core_axis_name="c", subcore_axis_name="s", num_cores=NC, num_subcores=NS
)


def _deg_body(dst_hbm, out_hbm, didx, ones_v, zb_v, dacc, sem):
    c = lax.axis_index("c")
    s = lax.axis_index("s")
    wid = s * NC + c

    @pl.loop(0, CHN // 16)
    def _(j):
        ones_v[pl.ds(j * 16, 16)] = jnp.ones((16,), f32)

    @pl.loop(0, DPT // 16)
    def _(j):
        zb_v[pl.ds(j * 16, 16)] = jnp.zeros((16,), f32)

    pltpu.sync_copy(zb_v, dacc.at[pl.ds(s * DPT, DPT)])
    pltpu.sync_copy(dst_hbm.at[wid], didx)
    plsc.subcore_barrier()

    @pl.loop(0, NCH)
    def _(i):
        pltpu.sync_copy(ones_v, dacc.at[didx.at[i]], add=True)

    plsc.subcore_barrier()
    pltpu.sync_copy(dacc.at[pl.ds(s * DPT, DPT)],
                    out_hbm.at[c, pl.ds(s * DPT, DPT)])


@jax.jit
def _sc_degree(dst3):
    return pl.kernel(
        _deg_body,
        out_type=jax.ShapeDtypeStruct((NC, NPAD), f32),
        mesh=_mesh,
        scratch_types=[
            pltpu.VMEM((NCH, CHN), jnp.int32),
            pltpu.VMEM((CHN,), f32),
            pltpu.VMEM((DPT,), f32),
            pltpu.VMEM_SHARED((NPAD,), f32),
            pltpu.SemaphoreType.DMA,
        ],
    )(dst3)


def _segsum_body(y_hbm, src_hbm, dst_hbm, out_hbm,
                 sring, didx, rows0, rows1, acc, semf, semg0, semg1):
    c = lax.axis_index("c")
    s = lax.axis_index("s")
    rows = (rows0, rows1)
    semg = (semg0, semg1)

    base = jnp.where(c == 0, s * CPT0, NS * CPT0 + s * CPT1)
    nch = jnp.where(c == 0, CPT0, CPT1)

    @pl.loop(0, CHN)
    def _(r):
        for j in range(D // 16):
            rows0[r, pl.ds(j * 16, 16)] = jnp.zeros((16,), f32)

    pltpu.sync_copy(dst_hbm.at[pl.ds(base, SLAB)], didx)

    @pl.loop(0, RPT // CHN)
    def _(k):
        pltpu.sync_copy(rows0, acc.at[pl.ds(s * RPT + k * CHN, CHN)])

    pltpu.sync_copy(rows0.at[pl.ds(0, RPT % CHN)],
                    acc.at[pl.ds(s * RPT + (RPT // CHN) * CHN, RPT % CHN)])

    plsc.subcore_barrier()

    def fetch(i, slot):
        pltpu.async_copy(src_hbm.at[pl.ds((base + i) * CHN, CHN)],
                         sring.at[slot], semf)

    def fetch_wait(slot):
        pltpu.make_async_copy(src_hbm.at[pl.ds(0, CHN)],
                              sring.at[slot], semf).wait()

    def gather(slot, k):
        pltpu.async_copy(y_hbm.at[sring.at[slot]], rows[k], semg[k])

    def gather_wait(slot, k):
        pltpu.make_async_copy(y_hbm.at[sring.at[slot]], rows[k],
                              semg[k]).wait()

    fetch(0, 0)
    fetch(1, 1)
    fetch_wait(0)
    gather(0, 0)

    @pl.loop(0, nch // 4)
    def _(g):
        for k in range(4):
            i = 4 * g + k
            nf = (k + 1) % 4
            nf2 = (k + 2) % 4

            if k == 0:
                @pl.when(jnp.logical_and(i > 0, i % SLAB == 0))
                def _():
                    off = pl.multiple_of(base + i, 8)
                    pltpu.sync_copy(dst_hbm.at[pl.ds(off, SLAB)], didx)

            @pl.when(i + 1 < nch)
            def _():
                fetch_wait(nf)
                gather(nf, (k + 1) % 2)

            @pl.when(i + 2 < nch)
            def _():
                fetch(i + 2, nf2)

            gather_wait(k % 4, k % 2)
            pltpu.sync_copy(rows[k % 2], acc.at[didx.at[i % SLAB]], add=True)

    plsc.subcore_barrier()
    pltpu.sync_copy(acc.at[pl.ds(s * RPT, RPT)],
                    out_hbm.at[c, pl.ds(s * RPT, RPT)])


@jax.jit
def _sc_segsum(y, src2, dst2):
    return pl.kernel(
        _segsum_body,
        out_type=jax.ShapeDtypeStruct((NC, NROWS, D), f32),
        mesh=_mesh,
        scratch_types=[
            pltpu.VMEM((4, CHN), jnp.int32),
            pltpu.VMEM((SLAB, CHN), jnp.int32),
            pltpu.VMEM((CHN, D), f32),
            pltpu.VMEM((CHN, D), f32),
            pltpu.VMEM_SHARED((NROWS, D), f32),
            pltpu.SemaphoreType.DMA,
            pltpu.SemaphoreType.DMA,
            pltpu.SemaphoreType.DMA,
        ],
    )(y, src2, dst2)


BR = 400
GRID = N // BR


def _s1_body(x_ref, w_ref, d0_ref, d1_ref, xw_ref, y_ref, dinv_ref):
    xw = jnp.dot(x_ref[...], w_ref[...], preferred_element_type=f32)
    dinv = lax.rsqrt(d0_ref[...] + d1_ref[...] + 1.0)
    xw_ref[...] = xw
    y_ref[...] = xw * dinv
    dinv_ref[...] = dinv


@jax.jit
def _tc_stage1(x, W1, d0, d1):
    return pl.pallas_call(
        _s1_body,
        grid=(GRID,),
        in_specs=[
            pl.BlockSpec((BR, D), lambda i: (i, 0)),
            pl.BlockSpec((D, D), lambda i: (0, 0)),
            pl.BlockSpec((BR, 1), lambda i: (i, 0)),
            pl.BlockSpec((BR, 1), lambda i: (i, 0)),
        ],
        out_specs=[
            pl.BlockSpec((BR, D), lambda i: (i, 0)),
            pl.BlockSpec((BR, D), lambda i: (i, 0)),
            pl.BlockSpec((BR, 1), lambda i: (i, 0)),
        ],
        out_shape=[
            jax.ShapeDtypeStruct((N, D), f32),
            jax.ShapeDtypeStruct((N, D), f32),
            jax.ShapeDtypeStruct((N, 1), f32),
        ],
    )(x, W1, d0, d1)


def _s2_body(s0_ref, s1_ref, xw1_ref, dinv_ref, b1_ref, w2_ref,
             xw2_ref, y2_ref):
    dinv = dinv_ref[...]
    h = (s0_ref[...] + s1_ref[...]) * dinv \
        + xw1_ref[...] * (dinv * dinv) + b1_ref[...]
    h = jnp.maximum(h, 0.0)
    xw2 = jnp.dot(h, w2_ref[...], preferred_element_type=f32)
    xw2_ref[...] = xw2
    y2_ref[...] = xw2 * dinv


@jax.jit
def _tc_stage2(s0, s1, xw1, dinv, b1, W2):
    return pl.pallas_call(
        _s2_body,
        grid=(GRID,),
        in_specs=[
            pl.BlockSpec((BR, D), lambda i: (i, 0)),
            pl.BlockSpec((BR, D), lambda i: (i, 0)),
            pl.BlockSpec((BR, D), lambda i: (i, 0)),
            pl.BlockSpec((BR, 1), lambda i: (i, 0)),
            pl.BlockSpec((1, D), lambda i: (0, 0)),
            pl.BlockSpec((D, D), lambda i: (0, 0)),
        ],
        out_specs=[
            pl.BlockSpec((BR, D), lambda i: (i, 0)),
            pl.BlockSpec((BR, D), lambda i: (i, 0)),
        ],
        out_shape=[
            jax.ShapeDtypeStruct((N, D), f32),
            jax.ShapeDtypeStruct((N, D), f32),
        ],
    )(s0, s1, xw1, dinv, b1, W2)


def _s3_body(s0_ref, s1_ref, xw2_ref, dinv_ref, b2_ref, out_ref):
    dinv = dinv_ref[...]
    out_ref[...] = (s0_ref[...] + s1_ref[...]) * dinv \
        + xw2_ref[...] * (dinv * dinv) + b2_ref[...]


@jax.jit
def _tc_stage3(s0, s1, xw2, dinv, b2):
    return pl.pallas_call(
        _s3_body,
        grid=(GRID,),
        in_specs=[
            pl.BlockSpec((BR, D), lambda i: (i, 0)),
            pl.BlockSpec((BR, D), lambda i: (i, 0)),
            pl.BlockSpec((BR, D), lambda i: (i, 0)),
            pl.BlockSpec((BR, 1), lambda i: (i, 0)),
            pl.BlockSpec((1, D), lambda i: (0, 0)),
        ],
        out_specs=pl.BlockSpec((BR, D), lambda i: (i, 0)),
        out_shape=jax.ShapeDtypeStruct((N, D), f32),
    )(s0, s1, xw2, dinv, b2)


def kernel(x, edge_index, W1, b1, W2, b2):
    src = edge_index[0].astype(jnp.int32)
    dst = edge_index[1].astype(jnp.int32)
    npad = EPAD - E
    src1d = jnp.concatenate([src, jnp.zeros((npad,), jnp.int32)])
    dst2 = jnp.concatenate(
        [dst, jnp.full((npad,), N, jnp.int32)]).reshape(TCHP, CHN)
    dst3 = dst2[:TCH].reshape(NW, NCH, CHN)

    degp = _sc_degree(dst3)
    d0 = degp[0, :N].reshape(N, 1)
    d1 = degp[1, :N].reshape(N, 1)

    xw1, y1, dinv = _tc_stage1(x, W1, d0, d1)

    s1p = _sc_segsum(y1, src1d, dst2)
    xw2, y2 = _tc_stage2(s1p[0, :N], s1p[1, :N], xw1, dinv,
                         b1.reshape(1, D), W2)

    s2p = _sc_segsum(y2, src1d, dst2)
    out = _tc_stage3(s2p[0, :N], s2p[1, :N], xw2, dinv, b2.reshape(1, D))
    return out


# speedup vs baseline: 1.1904x; 1.1337x over previous
import functools

import jax
import jax.numpy as jnp
from jax import lax
from jax.experimental import pallas as pl
from jax.experimental.pallas import tpu as pltpu
from jax.experimental.pallas import tpu_sc as plsc

N = 10000
E = 320000
D = 128

NC = 2
NS = 16
NW = NC * NS
EPW = E // NW
CH = 80
NCH = EPW // CH
NROWS = 10240
RPT = NROWS // NS
ZR = 128
DPT = 640
NPAD = NS * DPT

f32 = jnp.float32

_mesh = plsc.VectorSubcoreMesh(
    core_axis_name="c", subcore_axis_name="s", num_cores=NC, num_subcores=NS
)


def _deg_body(dst_hbm, out_hbm, idx_v, ones_v, zb_v, dacc, sem):
    c = lax.axis_index("c")
    s = lax.axis_index("s")
    wid = s * NC + c

    @pl.loop(0, CH // 16)
    def _(j):
        ones_v[pl.ds(j * 16, 16)] = jnp.ones((16,), f32)

    @pl.loop(0, DPT // 16)
    def _(j):
        zb_v[pl.ds(j * 16, 16)] = jnp.zeros((16,), f32)

    pltpu.sync_copy(zb_v, dacc.at[pl.ds(s * DPT, DPT)])
    plsc.subcore_barrier()

    base = wid * EPW

    @pl.loop(0, NCH)
    def _(i):
        pltpu.sync_copy(dst_hbm.at[pl.ds(base + i * CH, CH)], idx_v)
        pltpu.sync_copy(ones_v, dacc.at[idx_v], add=True)

    plsc.subcore_barrier()
    pltpu.sync_copy(dacc.at[pl.ds(s * DPT, DPT)],
                    out_hbm.at[c, pl.ds(s * DPT, DPT)])


@jax.jit
def _sc_degree(dst):
    return pl.kernel(
        _deg_body,
        out_type=jax.ShapeDtypeStruct((NC, NPAD), f32),
        mesh=_mesh,
        scratch_types=[
            pltpu.VMEM((CH,), jnp.int32),
            pltpu.VMEM((CH,), f32),
            pltpu.VMEM((DPT,), f32),
            pltpu.VMEM_SHARED((NPAD,), f32),
            pltpu.SemaphoreType.DMA,
        ],
    )(dst)


def _segsum_body(y_hbm, src_hbm, dst_hbm, out_hbm,
                 sidx, didx, rows, zb, acc, sem):
    c = lax.axis_index("c")
    s = lax.axis_index("s")
    wid = s * NC + c

    @pl.loop(0, ZR)
    def _(r):
        for j in range(D // 16):
            zb[r, pl.ds(j * 16, 16)] = jnp.zeros((16,), f32)

    @pl.loop(0, RPT // ZR)
    def _(k):
        pltpu.sync_copy(zb, acc.at[pl.ds(s * RPT + k * ZR, ZR)])

    plsc.subcore_barrier()

    base = wid * EPW

    @pl.loop(0, NCH)
    def _(i):
        pltpu.sync_copy(src_hbm.at[pl.ds(base + i * CH, CH)], sidx)
        pltpu.sync_copy(dst_hbm.at[pl.ds(base + i * CH, CH)], didx)
        pltpu.async_copy(y_hbm.at[sidx], rows, sem).wait()
        pltpu.sync_copy(rows, acc.at[didx], add=True)

    plsc.subcore_barrier()
    pltpu.sync_copy(acc.at[pl.ds(s * RPT, RPT)],
                    out_hbm.at[c, pl.ds(s * RPT, RPT)])


@jax.jit
def _sc_segsum(y, src, dst):
    return pl.kernel(
        _segsum_body,
        out_type=jax.ShapeDtypeStruct((NC, NROWS, D), f32),
        mesh=_mesh,
        scratch_types=[
            pltpu.VMEM((CH,), jnp.int32),
            pltpu.VMEM((CH,), jnp.int32),
            pltpu.VMEM((CH, D), f32),
            pltpu.VMEM((ZR, D), f32),
            pltpu.VMEM_SHARED((NROWS, D), f32),
            pltpu.SemaphoreType.DMA,
        ],
    )(y, src, dst)


BR = 400
GRID = N // BR


def _s1_body(x_ref, w_ref, d0_ref, d1_ref, xw_ref, y_ref, dinv_ref):
    xw = jnp.dot(x_ref[...], w_ref[...], preferred_element_type=f32)
    dinv = lax.rsqrt(d0_ref[...] + d1_ref[...] + 1.0)
    xw_ref[...] = xw
    y_ref[...] = xw * dinv
    dinv_ref[...] = dinv


@jax.jit
def _tc_stage1(x, W1, d0, d1):
    return pl.pallas_call(
        _s1_body,
        grid=(GRID,),
        in_specs=[
            pl.BlockSpec((BR, D), lambda i: (i, 0)),
            pl.BlockSpec((D, D), lambda i: (0, 0)),
            pl.BlockSpec((BR, 1), lambda i: (i, 0)),
            pl.BlockSpec((BR, 1), lambda i: (i, 0)),
        ],
        out_specs=[
            pl.BlockSpec((BR, D), lambda i: (i, 0)),
            pl.BlockSpec((BR, D), lambda i: (i, 0)),
            pl.BlockSpec((BR, 1), lambda i: (i, 0)),
        ],
        out_shape=[
            jax.ShapeDtypeStruct((N, D), f32),
            jax.ShapeDtypeStruct((N, D), f32),
            jax.ShapeDtypeStruct((N, 1), f32),
        ],
    )(x, W1, d0, d1)


def _s2_body(s0_ref, s1_ref, xw1_ref, dinv_ref, b1_ref, w2_ref,
             xw2_ref, y2_ref):
    dinv = dinv_ref[...]
    h = (s0_ref[...] + s1_ref[...]) * dinv \
        + xw1_ref[...] * (dinv * dinv) + b1_ref[...]
    h = jnp.maximum(h, 0.0)
    xw2 = jnp.dot(h, w2_ref[...], preferred_element_type=f32)
    xw2_ref[...] = xw2
    y2_ref[...] = xw2 * dinv


@jax.jit
def _tc_stage2(s0, s1, xw1, dinv, b1, W2):
    return pl.pallas_call(
        _s2_body,
        grid=(GRID,),
        in_specs=[
            pl.BlockSpec((BR, D), lambda i: (i, 0)),
            pl.BlockSpec((BR, D), lambda i: (i, 0)),
            pl.BlockSpec((BR, D), lambda i: (i, 0)),
            pl.BlockSpec((BR, 1), lambda i: (i, 0)),
            pl.BlockSpec((1, D), lambda i: (0, 0)),
            pl.BlockSpec((D, D), lambda i: (0, 0)),
        ],
        out_specs=[
            pl.BlockSpec((BR, D), lambda i: (i, 0)),
            pl.BlockSpec((BR, D), lambda i: (i, 0)),
        ],
        out_shape=[
            jax.ShapeDtypeStruct((N, D), f32),
            jax.ShapeDtypeStruct((N, D), f32),
        ],
    )(s0, s1, xw1, dinv, b1, W2)


def _s3_body(s0_ref, s1_ref, xw2_ref, dinv_ref, b2_ref, out_ref):
    dinv = dinv_ref[...]
    out_ref[...] = (s0_ref[...] + s1_ref[...]) * dinv \
        + xw2_ref[...] * (dinv * dinv) + b2_ref[...]


@jax.jit
def _tc_stage3(s0, s1, xw2, dinv, b2):
    return pl.pallas_call(
        _s3_body,
        grid=(GRID,),
        in_specs=[
            pl.BlockSpec((BR, D), lambda i: (i, 0)),
            pl.BlockSpec((BR, D), lambda i: (i, 0)),
            pl.BlockSpec((BR, D), lambda i: (i, 0)),
            pl.BlockSpec((BR, 1), lambda i: (i, 0)),
            pl.BlockSpec((1, D), lambda i: (0, 0)),
        ],
        out_specs=pl.BlockSpec((BR, D), lambda i: (i, 0)),
        out_shape=jax.ShapeDtypeStruct((N, D), f32),
    )(s0, s1, xw2, dinv, b2)


def kernel(x, edge_index, W1, b1, W2, b2):
    src = edge_index[0].astype(jnp.int32)
    dst = edge_index[1].astype(jnp.int32)

    degp = _sc_degree(dst)
    d0 = degp[0, :N].reshape(N, 1)
    d1 = degp[1, :N].reshape(N, 1)

    xw1, y1, dinv = _tc_stage1(x, W1, d0, d1)

    s1p = _sc_segsum(y1, src, dst)
    xw2, y2 = _tc_stage2(s1p[0, :N], s1p[1, :N], xw1, dinv,
                         b1.reshape(1, D), W2)

    s2p = _sc_segsum(y2, src, dst)
    out = _tc_stage3(s2p[0, :N], s2p[1, :N], xw2, dinv, b2.reshape(1, D))
    return out
